# baseline (speedup 1.0000x reference)
import jax
import jax.numpy as jnp
from jax import lax
from jax.experimental import pallas as pl
from jax.experimental.pallas import tpu as pltpu

B = 8
C = 192
C2 = 2 * C
R = C2 // 16
E = 16
K = 2
TEMP = 2.0
H = 224
S = H * H
ROWS = B * C
RB = 32
NRB = ROWS // RB
BPB = C // RB
EPS = 1e-5


def _gelu(t):
    return 0.5 * t * (1.0 + jax.lax.erf(t * (1.0 / jnp.sqrt(2.0))))


def _fused_kernel(x_ref, w1t_ref, b1_ref, bn1g_ref, bn1b_ref,
                  caw1t_ref, cab1_ref, caw2t_ref, cab2_ref,
                  w2t_ref, b2_ref, bn2g_ref, bn2b_ref,
                  w3t_ref, b3_ref,
                  idx_ref, val_ref, acc_ref):
    i = pl.program_id(0)

    @pl.when(i == 0)
    def _init():
        acc_ref[...] = jnp.zeros_like(acc_ref)

    s = jnp.sum(x_ref[...], axis=1)
    rowsum = jnp.sum(s, axis=1, keepdims=True)
    rep = jnp.concatenate([rowsum] * BPB, axis=0)
    ioc = lax.broadcasted_iota(jnp.int32, (C, 1), 0)
    q = (i % BPB) * RB
    sel = jnp.where((ioc >= q) & (ioc < q + RB), rep, 0.0)
    iob = lax.broadcasted_iota(jnp.int32, (B, 1), 0)
    oneh = (iob == i // BPB).astype(jnp.float32)
    acc_ref[...] += lax.dot_general(
        oneh, sel, (((1,), (1,)), ((), ())), preferred_element_type=jnp.float32)

    @pl.when(i == NRB - 1)
    def _finish():
        rs = 1.0 / jnp.sqrt(1.0 + EPS)
        g = acc_ref[...] * (1.0 / S)
        h = jnp.dot(g, w1t_ref[...], preferred_element_type=jnp.float32) + b1_ref[...]
        h = h * (bn1g_ref[...] * rs) + bn1b_ref[...]
        h = _gelu(h)
        t = _gelu(jnp.dot(h, caw1t_ref[...], preferred_element_type=jnp.float32) + cab1_ref[...])
        fc = jnp.dot(t, caw2t_ref[...], preferred_element_type=jnp.float32) + cab2_ref[...]
        att = jax.nn.sigmoid(2.0 * fc)
        hh = h * att
        h2 = jnp.dot(hh, w2t_ref[...], preferred_element_type=jnp.float32) + b2_ref[...]
        h2 = h2 * (bn2g_ref[...] * rs) + bn2b_ref[...]
        h2 = _gelu(h2)
        scores = jnp.dot(h2, w3t_ref[...], preferred_element_type=jnp.float32) + b3_ref[...]

        ids = lax.broadcasted_iota(jnp.int32, (B, E), 1)
        m1 = jnp.max(scores, axis=1, keepdims=True)
        i1 = jnp.min(jnp.where(scores == m1, ids, E), axis=1, keepdims=True)
        masked = jnp.where(ids == i1, -jnp.inf, scores)
        m2 = jnp.max(masked, axis=1, keepdims=True)
        i2 = jnp.min(jnp.where(masked == m2, ids, E), axis=1, keepdims=True)
        v1 = 1.0 / (1.0 + jnp.exp((m2 - m1) / TEMP))
        v2 = 1.0 - v1
        col = lax.broadcasted_iota(jnp.int32, (B, K), 1)
        idx_ref[...] = jnp.where(col == 0, i1, i2)
        val_ref[...] = jnp.where(col == 0, v1, v2)


def kernel(x, w1, b1, bn1_g, bn1_b, ca_w1, ca_b1, ca_w2, ca_b2, w2, b2, bn2_g, bn2_b, w3, b3):
    x3 = x.reshape(ROWS, H, H)
    row = lambda v: v.reshape(1, -1)
    full = lambda shp: pl.BlockSpec(shp, lambda i: (0,) * len(shp))
    idx, val = pl.pallas_call(
        _fused_kernel,
        grid=(NRB,),
        in_specs=[
            pl.BlockSpec((RB, H, H), lambda i: (i, 0, 0)),
            full((C, C2)), full((1, C2)), full((1, C2)), full((1, C2)),
            full((C2, R)), full((1, R)), full((R, C2)), full((1, C2)),
            full((C2, C)), full((1, C)), full((1, C)), full((1, C)),
            full((C, E)), full((1, E)),
        ],
        out_specs=[full((B, K)), full((B, K))],
        out_shape=[
            jax.ShapeDtypeStruct((B, K), jnp.int32),
            jax.ShapeDtypeStruct((B, K), jnp.float32),
        ],
        scratch_shapes=[pltpu.VMEM((B, C), jnp.float32)],
        compiler_params=pltpu.CompilerParams(
            dimension_semantics=("arbitrary",),
        ),
    )(x3, w1.T, row(b1), row(bn1_g), row(bn1_b),
      ca_w1.T, row(ca_b1), ca_w2.T, row(ca_b2),
      w2.T, row(b2), row(bn2_g), row(bn2_b),
      w3.T, row(b3))
    return idx, val
